# baseline (speedup 1.0000x reference)
import functools

import jax
import jax.numpy as jnp
from jax import lax
from jax.experimental import pallas as pl
from jax.experimental.pallas import tpu as pltpu
from jax.experimental.pallas import tpu_sc as plsc

CHUNK = 128
NBUF = 3


@functools.lru_cache(maxsize=None)
def _make(n_idx, d, nc, ns):
    nw = nc * ns
    per_w = n_idx // nw
    T = per_w // CHUNK
    assert per_w % CHUNK == 0 and (T - 2) % NBUF == 0 and T >= 8
    n_groups = (T - 2) // NBUF
    mesh = plsc.VectorSubcoreMesh(core_axis_name="c", subcore_axis_name="s")

    @functools.partial(
        pl.kernel,
        mesh=mesh,
        out_type=jax.ShapeDtypeStruct((n_idx, d), jnp.float32),
        scratch_types=[
            [pltpu.VMEM((CHUNK,), jnp.int32) for _ in range(NBUF)],
            [pltpu.VMEM((CHUNK,), jnp.int32) for _ in range(NBUF)],
            [pltpu.VMEM((CHUNK + 16,), jnp.int32) for _ in range(NBUF)],
            [pltpu.VMEM((CHUNK, 128), jnp.float32) for _ in range(NBUF)],
            [pltpu.VMEM((CHUNK, d), jnp.float32) for _ in range(NBUF)],
            [pltpu.SemaphoreType.DMA for _ in range(NBUF)],
            [pltpu.SemaphoreType.DMA for _ in range(NBUF)],
            [pltpu.SemaphoreType.DMA for _ in range(NBUF)],
        ],
        compiler_params=pltpu.CompilerParams(use_tc_tiling_on_sc=True),
    )
    def k(idx_hbm, pairs_hbm, out_hbm, idx_c, pidx, offs_c, rows128, rows64,
          ia, ga, oa):
        wid = lax.axis_index("s") * nc + lax.axis_index("c")
        base = wid * per_w

        def start_idx(t, b):
            pltpu.async_copy(idx_hbm.at[pl.ds(base + t * CHUNK, CHUNK)], idx_c[b], ia[b])

        def wait_idx(b):
            pltpu.make_async_copy(idx_hbm.at[pl.ds(base, CHUNK)], idx_c[b], ia[b]).wait()

        def prep(t, b):
            del t
            for v in range(CHUNK // 16):
                s = pl.ds(v * 16, 16)
                iv = idx_c[b][s]
                pidx[b][s] = lax.shift_right_logical(iv, 1)
                offs_c[b][s] = (iv & 1) * 64

        def start_gather(b):
            pltpu.async_copy(pairs_hbm.at[pidx[b]], rows128[b], ga[b])

        def wait_gather(b):
            pltpu.make_async_copy(pairs_hbm.at[pidx[b]], rows128[b], ga[b]).wait()

        def select(b):
            @pl.loop(0, CHUNK, unroll=2)
            def _(i):
                off = offs_c[b][pl.ds(i, 16)][0]
                for j in range(d // 16):
                    rows64[b][i, pl.ds(j * 16, 16)] = rows128[b][i, pl.ds(off + j * 16, 16)]

        def start_out(t, b):
            pltpu.async_copy(rows64[b], out_hbm.at[pl.ds(base + t * CHUNK, CHUNK)], oa[b])

        def wait_out(b):
            pltpu.make_async_copy(rows64[b], out_hbm.at[pl.ds(base, CHUNK)], oa[b]).wait()

        def complete_prev(t, b1, prefetch):
            wait_gather(b1)
            select(b1)
            start_out(t - 1, b1)
            if prefetch:
                start_idx(t + 2, b1)

        start_idx(0, 0)
        start_idx(1, 1)
        wait_idx(0)
        prep(0, 0)
        start_gather(0)
        start_idx(2, 2)
        wait_idx(1)
        prep(1, 1)
        start_gather(1)
        complete_prev(1, 0, True)

        @pl.loop(0, n_groups)
        def _(g):
            t0 = g * NBUF + 2
            for u in range(NBUF):
                t = t0 + u
                b = (2 + u) % NBUF
                b1 = (b + NBUF - 1) % NBUF
                wait_idx(b)
                prep(t, b)
                if u == 0:
                    @pl.when(t >= NBUF)
                    def _():
                        wait_out(b)
                else:
                    wait_out(b)
                start_gather(b)

                @pl.when(t + 2 <= T - 1)
                def _():
                    complete_prev(t, b1, True)

                @pl.when(t + 2 > T - 1)
                def _():
                    complete_prev(t, b1, False)

        bl = (T - 1) % NBUF
        wait_gather(bl)
        select(bl)
        start_out(T - 1, bl)
        for b in range(NBUF):
            wait_out(b)

    return k


@jax.jit
def kernel(x, table):
    idx = x.reshape(-1).astype(jnp.int32)
    pairs = table.reshape(table.shape[0] // 2, 2 * table.shape[1])
    info = plsc.get_sparse_core_info()
    out = _make(idx.shape[0], table.shape[1], info.num_cores, info.num_subcores)(idx, pairs)
    return out.reshape(x.shape + (table.shape[1],))


# speedup vs baseline: 1.2057x; 1.2057x over previous
import functools

import jax
import jax.numpy as jnp
from jax import lax
from jax.experimental import pallas as pl
from jax.experimental.pallas import tpu as pltpu
from jax.experimental.pallas import tpu_sc as plsc

CHUNK = 128
NBUF = 3


@functools.lru_cache(maxsize=None)
def _make(n_idx, d, nc, ns):
    nw = nc * ns
    per_w = n_idx // nw
    T = per_w // CHUNK
    assert per_w % CHUNK == 0 and (T - 2) % NBUF == 0 and T >= 8
    n_groups = (T - 2) // NBUF
    mesh = plsc.VectorSubcoreMesh(core_axis_name="c", subcore_axis_name="s")

    @functools.partial(
        pl.kernel,
        mesh=mesh,
        out_type=jax.ShapeDtypeStruct((n_idx, d), jnp.float32),
        scratch_types=[
            [pltpu.VMEM((CHUNK,), jnp.int32) for _ in range(NBUF)],
            [pltpu.VMEM((CHUNK,), jnp.int32) for _ in range(NBUF)],
            [pltpu.VMEM((CHUNK + 16,), jnp.int32) for _ in range(NBUF)],
            [pltpu.VMEM((CHUNK, 128), jnp.float32) for _ in range(NBUF)],
            [pltpu.VMEM((CHUNK, d), jnp.float32) for _ in range(NBUF)],
            [pltpu.SemaphoreType.DMA for _ in range(NBUF)],
            [pltpu.SemaphoreType.DMA for _ in range(NBUF)],
            [pltpu.SemaphoreType.DMA for _ in range(NBUF)],
        ],
        compiler_params=pltpu.CompilerParams(use_tc_tiling_on_sc=True),
    )
    def k(idx_hbm, pairs_hbm, out_hbm, idx_c, pidx, offs_c, rows128, rows64,
          ia, ga, oa):
        wid = lax.axis_index("s") * nc + lax.axis_index("c")
        base = wid * per_w

        def start_idx(t, b):
            pltpu.async_copy(idx_hbm.at[pl.ds(base + t * CHUNK, CHUNK)], idx_c[b], ia[b])

        def wait_idx(b):
            pltpu.make_async_copy(idx_hbm.at[pl.ds(base, CHUNK)], idx_c[b], ia[b]).wait()

        def prep(t, b):
            del t
            for v in range(CHUNK // 16):
                s = pl.ds(v * 16, 16)
                iv = idx_c[b][s]
                pidx[b][s] = lax.shift_right_logical(iv, 1)
                offs_c[b][s] = (iv & 1) * 64

        def start_gather(b):
            pltpu.async_copy(pairs_hbm.at[pidx[b]], rows128[b], ga[b])

        def wait_gather(b):
            pltpu.make_async_copy(pairs_hbm.at[pidx[b]], rows128[b], ga[b]).wait()

        def select(b):
            @pl.loop(0, CHUNK // 16)
            def _(g):
                i0 = g * 16
                ov = offs_c[b][pl.ds(i0, 16)]
                for l in range(16):
                    off = ov[l]
                    i = i0 + l
                    for j in range(d // 16):
                        rows64[b][i, pl.ds(j * 16, 16)] = rows128[b][i, pl.ds(off + j * 16, 16)]

        def start_out(t, b):
            pltpu.async_copy(rows64[b], out_hbm.at[pl.ds(base + t * CHUNK, CHUNK)], oa[b])

        def wait_out(b):
            pltpu.make_async_copy(rows64[b], out_hbm.at[pl.ds(base, CHUNK)], oa[b]).wait()

        def complete_prev(t, b1, prefetch):
            wait_gather(b1)
            select(b1)
            start_out(t - 1, b1)
            if prefetch:
                start_idx(t + 2, b1)

        start_idx(0, 0)
        start_idx(1, 1)
        wait_idx(0)
        prep(0, 0)
        start_gather(0)
        start_idx(2, 2)
        wait_idx(1)
        prep(1, 1)
        start_gather(1)
        complete_prev(1, 0, True)

        @pl.loop(0, n_groups)
        def _(g):
            t0 = g * NBUF + 2
            for u in range(NBUF):
                t = t0 + u
                b = (2 + u) % NBUF
                b1 = (b + NBUF - 1) % NBUF
                wait_idx(b)
                prep(t, b)
                if u == 0:
                    @pl.when(t >= NBUF)
                    def _():
                        wait_out(b)
                else:
                    wait_out(b)
                start_gather(b)

                @pl.when(t + 2 <= T - 1)
                def _():
                    complete_prev(t, b1, True)

                @pl.when(t + 2 > T - 1)
                def _():
                    complete_prev(t, b1, False)

        bl = (T - 1) % NBUF
        wait_gather(bl)
        select(bl)
        start_out(T - 1, bl)
        for b in range(NBUF):
            wait_out(b)

    return k


@jax.jit
def kernel(x, table):
    idx = x.reshape(-1).astype(jnp.int32)
    pairs = table.reshape(table.shape[0] // 2, 2 * table.shape[1])
    info = plsc.get_sparse_core_info()
    out = _make(idx.shape[0], table.shape[1], info.num_cores, info.num_subcores)(idx, pairs)
    return out.reshape(x.shape + (table.shape[1],))


# speedup vs baseline: 1.3078x; 1.0847x over previous
import functools

import jax
import jax.numpy as jnp
from jax import lax
from jax.experimental import pallas as pl
from jax.experimental.pallas import tpu as pltpu
from jax.experimental.pallas import tpu_sc as plsc

CHUNK = 128
NBUF = 3


@functools.lru_cache(maxsize=None)
def _make(n_idx, d, nc, ns):
    nw = nc * ns
    per_w = n_idx // nw
    T = per_w // CHUNK
    assert per_w % CHUNK == 0 and (T - 2) % NBUF == 0 and T >= 8
    n_groups = (T - 2) // NBUF
    mesh = plsc.VectorSubcoreMesh(core_axis_name="c", subcore_axis_name="s")

    @functools.partial(
        pl.kernel,
        mesh=mesh,
        out_type=jax.ShapeDtypeStruct((n_idx, d), jnp.float32),
        scratch_types=[
            [pltpu.VMEM((CHUNK,), jnp.int32) for _ in range(NBUF)],
            [pltpu.VMEM((CHUNK,), jnp.int32) for _ in range(NBUF)],
            [pltpu.VMEM((CHUNK, 128), jnp.float32) for _ in range(NBUF)],
            [pltpu.VMEM((CHUNK, d), jnp.float32) for _ in range(NBUF)],
            [pltpu.SemaphoreType.DMA for _ in range(NBUF)],
            [pltpu.SemaphoreType.DMA for _ in range(NBUF)],
            [pltpu.SemaphoreType.DMA for _ in range(NBUF)],
        ],
        compiler_params=pltpu.CompilerParams(use_tc_tiling_on_sc=True),
    )
    def k(idx_hbm, pairs_hbm, out_hbm, idx_c, pidx, rows128, rows64,
          ia, ga, oa):
        wid = lax.axis_index("s") * nc + lax.axis_index("c")
        base = wid * per_w

        def start_idx(t, b):
            pltpu.async_copy(idx_hbm.at[pl.ds(base + t * CHUNK, CHUNK)], idx_c[b], ia[b])

        def wait_idx(b):
            pltpu.make_async_copy(idx_hbm.at[pl.ds(base, CHUNK)], idx_c[b], ia[b]).wait()

        def prep(t, b):
            del t
            for v in range(CHUNK // 16):
                s = pl.ds(v * 16, 16)
                pidx[b][s] = lax.shift_right_logical(idx_c[b][s], 1)

        def start_gather(b):
            pltpu.async_copy(pairs_hbm.at[pidx[b]], rows128[b], ga[b])

        def wait_gather(b):
            pltpu.make_async_copy(pairs_hbm.at[pidx[b]], rows128[b], ga[b]).wait()

        def select(b):
            @pl.loop(0, CHUNK // 16)
            def _(g):
                i0 = g * 16
                v = idx_c[b][pl.ds(i0, 16)]
                for l in range(16):
                    off = (v[l] & 1) * 64
                    i = i0 + l
                    for j in range(d // 16):
                        rows64[b][i, pl.ds(j * 16, 16)] = rows128[b][i, pl.ds(off + j * 16, 16)]

        def start_out(t, b):
            pltpu.async_copy(rows64[b], out_hbm.at[pl.ds(base + t * CHUNK, CHUNK)], oa[b])

        def wait_out(b):
            pltpu.make_async_copy(rows64[b], out_hbm.at[pl.ds(base, CHUNK)], oa[b]).wait()

        def complete_prev(t, b1, prefetch):
            wait_gather(b1)
            select(b1)
            start_out(t - 1, b1)
            if prefetch:
                start_idx(t + 2, b1)

        start_idx(0, 0)
        start_idx(1, 1)
        wait_idx(0)
        prep(0, 0)
        start_gather(0)
        start_idx(2, 2)
        wait_idx(1)
        prep(1, 1)
        start_gather(1)
        complete_prev(1, 0, True)

        @pl.loop(0, n_groups)
        def _(g):
            t0 = g * NBUF + 2
            for u in range(NBUF):
                t = t0 + u
                b = (2 + u) % NBUF
                b1 = (b + NBUF - 1) % NBUF
                wait_idx(b)
                prep(t, b)
                if u == 0:
                    @pl.when(t >= NBUF)
                    def _():
                        wait_out(b)
                else:
                    wait_out(b)
                start_gather(b)

                @pl.when(t + 2 <= T - 1)
                def _():
                    complete_prev(t, b1, True)

                @pl.when(t + 2 > T - 1)
                def _():
                    complete_prev(t, b1, False)

        bl = (T - 1) % NBUF
        wait_gather(bl)
        select(bl)
        start_out(T - 1, bl)
        for b in range(NBUF):
            wait_out(b)

    return k


@jax.jit
def kernel(x, table):
    idx = x.reshape(-1).astype(jnp.int32)
    pairs = table.reshape(table.shape[0] // 2, 2 * table.shape[1])
    info = plsc.get_sparse_core_info()
    out = _make(idx.shape[0], table.shape[1], info.num_cores, info.num_subcores)(idx, pairs)
    return out.reshape(x.shape + (table.shape[1],))


# speedup vs baseline: 1.3782x; 1.0538x over previous
import functools

import jax
import jax.numpy as jnp
from jax import lax
from jax.experimental import pallas as pl
from jax.experimental.pallas import tpu as pltpu
from jax.experimental.pallas import tpu_sc as plsc

CHUNK = 128
NBUF = 3


@functools.lru_cache(maxsize=None)
def _make(n_idx, d, nc, ns):
    nw = nc * ns
    per_w = n_idx // nw
    T = per_w // CHUNK
    assert per_w % CHUNK == 0 and (T - 2) % NBUF == 0 and T >= 8
    n_groups = (T - 2) // NBUF
    mesh = plsc.VectorSubcoreMesh(core_axis_name="c", subcore_axis_name="s")

    @functools.partial(
        pl.kernel,
        mesh=mesh,
        out_type=jax.ShapeDtypeStruct((n_idx, d), jnp.float32),
        scratch_types=[
            [pltpu.VMEM((CHUNK,), jnp.int32) for _ in range(NBUF)],
            [pltpu.VMEM((CHUNK,), jnp.int32) for _ in range(NBUF)],
            [pltpu.VMEM((CHUNK, 128), jnp.float32) for _ in range(NBUF)],
            [pltpu.VMEM((CHUNK, d), jnp.float32) for _ in range(NBUF)],
            [pltpu.SemaphoreType.DMA for _ in range(NBUF)],
            [pltpu.SemaphoreType.DMA for _ in range(NBUF)],
            [pltpu.SemaphoreType.DMA for _ in range(NBUF)],
        ],
        compiler_params=pltpu.CompilerParams(use_tc_tiling_on_sc=True),
    )
    def k(idx_hbm, pairs_hbm, out_hbm, idx_c, pidx, rows128, rows64,
          ia, ga, oa):
        wid = lax.axis_index("s") * nc + lax.axis_index("c")
        base = wid * per_w

        def start_idx(t, b):
            pltpu.async_copy(idx_hbm.at[pl.ds(base + t * CHUNK, CHUNK)], idx_c[b], ia[b])

        def wait_idx(b):
            pltpu.make_async_copy(idx_hbm.at[pl.ds(base, CHUNK)], idx_c[b], ia[b]).wait()

        def prep(t, b):
            del t
            for v in range(CHUNK // 16):
                s = pl.ds(v * 16, 16)
                pidx[b][s] = lax.shift_right_logical(idx_c[b][s], 1)

        def start_gather(b):
            pltpu.async_copy(pairs_hbm.at[pidx[b]], rows128[b], ga[b])

        def wait_gather(b):
            pltpu.make_async_copy(pairs_hbm.at[pidx[b]], rows128[b], ga[b]).wait()

        def select(b):
            @pl.loop(0, CHUNK)
            def _(i):
                for j in range(d // 16):
                    rows64[b][i, pl.ds(j * 16, 16)] = rows128[b][i, pl.ds(j * 16, 16)]

            @pl.loop(0, CHUNK // 16)
            def _(g):
                i0 = g * 16
                v = idx_c[b][pl.ds(i0, 16)]
                for l in range(16):
                    i = i0 + l

                    @pl.when((v[l] & 1) != 0)
                    def _():
                        for j in range(d // 16):
                            rows64[b][i, pl.ds(j * 16, 16)] = rows128[b][i, pl.ds(64 + j * 16, 16)]

        def start_out(t, b):
            pltpu.async_copy(rows64[b], out_hbm.at[pl.ds(base + t * CHUNK, CHUNK)], oa[b])

        def wait_out(b):
            pltpu.make_async_copy(rows64[b], out_hbm.at[pl.ds(base, CHUNK)], oa[b]).wait()

        def complete_prev(t, b1, prefetch):
            wait_gather(b1)
            select(b1)
            start_out(t - 1, b1)
            if prefetch:
                start_idx(t + 2, b1)

        start_idx(0, 0)
        start_idx(1, 1)
        wait_idx(0)
        prep(0, 0)
        start_gather(0)
        start_idx(2, 2)
        wait_idx(1)
        prep(1, 1)
        start_gather(1)
        complete_prev(1, 0, True)

        @pl.loop(0, n_groups)
        def _(g):
            t0 = g * NBUF + 2
            for u in range(NBUF):
                t = t0 + u
                b = (2 + u) % NBUF
                b1 = (b + NBUF - 1) % NBUF
                wait_idx(b)
                prep(t, b)
                if u == 0:
                    @pl.when(t >= NBUF)
                    def _():
                        wait_out(b)
                else:
                    wait_out(b)
                start_gather(b)

                @pl.when(t + 2 <= T - 1)
                def _():
                    complete_prev(t, b1, True)

                @pl.when(t + 2 > T - 1)
                def _():
                    complete_prev(t, b1, False)

        bl = (T - 1) % NBUF
        wait_gather(bl)
        select(bl)
        start_out(T - 1, bl)
        for b in range(NBUF):
            wait_out(b)

    return k


@jax.jit
def kernel(x, table):
    idx = x.reshape(-1).astype(jnp.int32)
    pairs = table.reshape(table.shape[0] // 2, 2 * table.shape[1])
    info = plsc.get_sparse_core_info()
    out = _make(idx.shape[0], table.shape[1], info.num_cores, info.num_subcores)(idx, pairs)
    return out.reshape(x.shape + (table.shape[1],))
